# Initial kernel scaffold; baseline (speedup 1.0000x reference)
import jax
import jax.numpy as jnp
from jax.experimental import pallas as pl


def kernel(x, edge_index, edges_weight, W1, b1, W2, b2):
    raise NotImplementedError("write your pallas kernel here")


# speedup vs baseline: 25.0848x; 25.0848x over previous
import functools

import jax
import jax.numpy as jnp
from jax import lax
from jax.experimental import pallas as pl
from jax.experimental.pallas import tpu as pltpu
from jax.experimental.pallas import tpu_sc as plsc

N = 10000
E = 320000
DF = 128
DH = 16
NC = 4

LANES = 16
EPR = 80
ROWS = E // EPR
NW = 32
RPT = ROWS // NW
NPT = N // 16

_mesh = plsc.VectorSubcoreMesh(core_axis_name="c", subcore_axis_name="s")
_sc_params = pltpu.CompilerParams(needs_layout_passes=False,
                                  use_tc_tiling_on_sc=False)


_GDN = lax.GatherDimensionNumbers(
    offset_dims=(), collapsed_slice_dims=(0,), start_index_map=(0,))


def _splat(vec16, i):
    idx = jnp.full((LANES, 1), i, dtype=jnp.int32)
    return lax.gather(vec16, idx, _GDN, slice_sizes=(1,),
                      mode=lax.GatherScatterMode.PROMISE_IN_BOUNDS)


@functools.partial(
    pl.kernel,
    mesh=_mesh,
    out_type=jax.ShapeDtypeStruct((NW, N), jnp.float32),
    compiler_params=_sc_params,
    scratch_types=[
        pltpu.VMEM((N,), jnp.float32),
        pltpu.VMEM((RPT, EPR), jnp.int32),
        pltpu.VMEM((RPT, EPR), jnp.float32),
    ],
)
def _sc_deg(col_hbm, ew_hbm, out_hbm, deg_v, col_v, ew_v):
    cid = lax.axis_index("c")
    sid = lax.axis_index("s")
    wid = cid * 16 + sid

    def zero(i, _):
        deg_v[pl.ds(i * LANES, LANES)] = jnp.zeros((LANES,), jnp.float32)
        return 0

    lax.fori_loop(0, N // LANES, zero, 0)

    pltpu.sync_copy(col_hbm.at[wid], col_v)
    pltpu.sync_copy(ew_hbm.at[wid], ew_v)

    def body(j, _):
        for k in range(EPR // LANES):
            c16 = col_v[j, pl.ds(k * LANES, LANES)]
            w16 = ew_v[j, pl.ds(k * LANES, LANES)]
            plsc.addupdate_scatter(deg_v, [c16], w16)
        return 0

    lax.fori_loop(0, RPT, body, 0)
    pltpu.sync_copy(deg_v, out_hbm.at[wid])


@functools.partial(
    pl.kernel,
    mesh=_mesh,
    out_type=jax.ShapeDtypeStruct((2, 16, NPT, DH), jnp.float32),
    compiler_params=_sc_params,
    scratch_types=[
        pltpu.VMEM((RPT, EPR), jnp.int32),
        pltpu.VMEM((RPT, EPR), jnp.int32),
        pltpu.VMEM((RPT, EPR), jnp.float32),
        pltpu.VMEM((EPR, DH), jnp.float32),
        pltpu.VMEM((NPT, DH), jnp.float32),
        pltpu.VMEM_SHARED((N, DH), jnp.float32),
        pltpu.SemaphoreType.DMA,
    ],
)
def _sc_prop(y_hbm, row_hbm, col_hbm, ew_hbm, out_hbm,
             row_v, col_v, ew_v, msg_v, stage_v, acc_sp, sem):
    cid = lax.axis_index("c")
    sid = lax.axis_index("s")
    wid = cid * 16 + sid

    def zero(i, _):
        stage_v[i, :] = jnp.zeros((LANES,), jnp.float32)
        return 0

    lax.fori_loop(0, NPT, zero, 0)
    pltpu.sync_copy(stage_v, acc_sp.at[pl.ds(sid * NPT, NPT)])
    plsc.subcore_barrier()

    pltpu.sync_copy(row_hbm.at[wid], row_v)
    pltpu.sync_copy(col_hbm.at[wid], col_v)
    pltpu.sync_copy(ew_hbm.at[wid], ew_v)

    def body(j, _):
        pltpu.async_copy(y_hbm.at[row_v.at[j]], msg_v, sem).wait()
        for k in range(EPR // LANES):
            w16 = ew_v[j, pl.ds(k * LANES, LANES)]
            for i in range(LANES):
                e = k * LANES + i
                msg_v[e, :] = msg_v[e, :] * _splat(w16, i)
        pltpu.sync_copy(msg_v, acc_sp.at[col_v.at[j]], add=True)
        return 0

    lax.fori_loop(0, RPT, body, 0)
    plsc.subcore_barrier()
    pltpu.sync_copy(acc_sp.at[pl.ds(sid * NPT, NPT)], out_hbm.at[cid, sid])


def _tc_prep_body(dpt_ref, x_ref, w1_ref, y1_ref, dinv_ref):
    deg = jnp.sum(dpt_ref[...], axis=1, keepdims=True) + 1.0
    dinv = lax.rsqrt(deg)
    dinv_ref[...] = dinv
    y1_ref[...] = jnp.dot(x_ref[...], w1_ref[...],
                          preferred_element_type=jnp.float32) * dinv


def _tc_mid_body(a0_ref, a1_ref, y1_ref, dinv_ref, b1_ref, w2_ref, y2_ref):
    dinv = dinv_ref[...]
    h = (a0_ref[...] + a1_ref[...] + y1_ref[...]) * dinv + b1_ref[...]
    h = jnp.maximum(h, 0.0)
    y2_ref[...] = jnp.dot(h, w2_ref[...],
                          preferred_element_type=jnp.float32) * dinv


def _tc_final_body(a0_ref, a1_ref, y2_ref, dinv_ref, b2_ref, out_ref):
    z = (a0_ref[...] + a1_ref[...] + y2_ref[...]) * dinv_ref[...] + b2_ref[...]
    m = jnp.max(z, axis=1, keepdims=True)
    lse = jnp.log(jnp.sum(jnp.exp(z - m), axis=1, keepdims=True)) + m
    out_ref[...] = z - lse


def kernel(x, edge_index, edges_weight, W1, b1, W2, b2):
    row2d = edge_index[0].reshape(NW, RPT, EPR)
    col2d = edge_index[1].reshape(NW, RPT, EPR)
    ew2d = edges_weight.reshape(NW, RPT, EPR)

    deg_part = _sc_deg(col2d, ew2d)
    dpt = deg_part.T

    y1, dinv = pl.pallas_call(
        _tc_prep_body,
        out_shape=(jax.ShapeDtypeStruct((N, DH), jnp.float32),
                   jax.ShapeDtypeStruct((N, 1), jnp.float32)),
    )(dpt, x, W1)

    acc = _sc_prop(y1, row2d, col2d, ew2d).reshape(2, N, DH)

    w2pad = jnp.zeros((DH, DH), jnp.float32).at[:, :NC].set(W2)
    b1r = b1.reshape(1, DH)
    y2 = pl.pallas_call(
        _tc_mid_body,
        out_shape=jax.ShapeDtypeStruct((N, DH), jnp.float32),
    )(acc[0], acc[1], y1, dinv, b1r, w2pad)

    acc2 = _sc_prop(y2, row2d, col2d, ew2d).reshape(2, N, DH)

    b2pad = jnp.full((1, DH), -1e30, jnp.float32).at[0, :NC].set(b2)
    outp = pl.pallas_call(
        _tc_final_body,
        out_shape=jax.ShapeDtypeStruct((N, DH), jnp.float32),
    )(acc2[0], acc2[1], y2, dinv, b2pad)

    return outp[:, :NC]


# speedup vs baseline: 46.9604x; 1.8721x over previous
import functools

import jax
import jax.numpy as jnp
from jax import lax
from jax.experimental import pallas as pl
from jax.experimental.pallas import tpu as pltpu
from jax.experimental.pallas import tpu_sc as plsc

N = 10000
E = 320000
DF = 128
DH = 16
NC = 4

LANES = 16
EPR = 80
ROWS = E // EPR
NW = 32
RPT = ROWS // NW
NPT = N // 16

_mesh = plsc.VectorSubcoreMesh(core_axis_name="c", subcore_axis_name="s")
_sc_params = pltpu.CompilerParams(needs_layout_passes=False,
                                  use_tc_tiling_on_sc=False)


_GDN = lax.GatherDimensionNumbers(
    offset_dims=(), collapsed_slice_dims=(0,), start_index_map=(0,))


def _splat(vec16, i):
    idx = jnp.full((LANES, 1), i, dtype=jnp.int32)
    return lax.gather(vec16, idx, _GDN, slice_sizes=(1,),
                      mode=lax.GatherScatterMode.PROMISE_IN_BOUNDS)


@functools.partial(
    pl.kernel,
    mesh=_mesh,
    out_type=jax.ShapeDtypeStruct((NW, N), jnp.float32),
    compiler_params=_sc_params,
    scratch_types=[
        pltpu.VMEM((N,), jnp.float32),
        pltpu.VMEM((RPT, EPR), jnp.int32),
        pltpu.VMEM((RPT, EPR), jnp.float32),
    ],
)
def _sc_deg(col_hbm, ew_hbm, out_hbm, deg_v, col_v, ew_v):
    cid = lax.axis_index("c")
    sid = lax.axis_index("s")
    wid = cid * 16 + sid

    def zero(i, _):
        deg_v[pl.ds(i * LANES, LANES)] = jnp.zeros((LANES,), jnp.float32)
        return 0

    lax.fori_loop(0, N // LANES, zero, 0)

    pltpu.sync_copy(col_hbm.at[wid], col_v)
    pltpu.sync_copy(ew_hbm.at[wid], ew_v)

    def body(j, _):
        for k in range(EPR // LANES):
            c16 = col_v[j, pl.ds(k * LANES, LANES)]
            w16 = ew_v[j, pl.ds(k * LANES, LANES)]
            plsc.addupdate_scatter(deg_v, [c16], w16)
        return 0

    lax.fori_loop(0, RPT, body, 0)
    pltpu.sync_copy(deg_v, out_hbm.at[wid])


NBUF = 5
MAIN = RPT // NBUF


@functools.partial(
    pl.kernel,
    mesh=_mesh,
    out_type=jax.ShapeDtypeStruct((2, 16, NPT, DH), jnp.float32),
    compiler_params=_sc_params,
    scratch_types=[
        pltpu.VMEM((RPT, EPR), jnp.int32),
        pltpu.VMEM((RPT, EPR), jnp.int32),
        pltpu.VMEM((RPT, EPR), jnp.float32),
        pltpu.VMEM((NPT, DH), jnp.float32),
        pltpu.VMEM_SHARED((N, DH), jnp.float32),
    ]
    + [pltpu.VMEM((EPR, DH), jnp.float32)] * (2 * NBUF)
    + [pltpu.SemaphoreType.DMA] * (2 * NBUF + 1),
)
def _sc_prop(y_hbm, row_hbm, col_hbm, ew_hbm, out_hbm,
             row_v, col_v, ew_v, stage_v, acc_sp, *bufs_and_sems):
    gbuf = bufs_and_sems[:NBUF]
    sbuf = bufs_and_sems[NBUF:2 * NBUF]
    gsem = bufs_and_sems[2 * NBUF:3 * NBUF]
    ssem = bufs_and_sems[3 * NBUF:4 * NBUF]
    insem = bufs_and_sems[4 * NBUF]
    cid = lax.axis_index("c")
    sid = lax.axis_index("s")
    wid = cid * 16 + sid

    pltpu.async_copy(row_hbm.at[wid], row_v, insem)
    pltpu.async_copy(col_hbm.at[wid], col_v, insem)
    pltpu.async_copy(ew_hbm.at[wid], ew_v, insem)

    def zero(i, _):
        stage_v[i, :] = jnp.zeros((LANES,), jnp.float32)
        return 0

    lax.fori_loop(0, NPT, zero, 0)
    pltpu.make_async_copy(row_hbm.at[wid], row_v, insem).wait()
    pltpu.make_async_copy(col_hbm.at[wid], col_v, insem).wait()
    pltpu.make_async_copy(ew_hbm.at[wid], ew_v, insem).wait()

    for t in range(NBUF):
        pltpu.async_copy(y_hbm.at[row_v.at[t]], gbuf[t], gsem[t])
    pltpu.sync_copy(stage_v, acc_sp.at[pl.ds(sid * NPT, NPT)])
    plsc.subcore_barrier()

    def mbody(m, _):
        for t in range(NBUF):
            j = m * NBUF + t
            pltpu.make_async_copy(y_hbm.at[row_v.at[j]], gbuf[t], gsem[t]).wait()

            @pl.when(m > 0)
            def _wait_scatter():
                pltpu.make_async_copy(
                    sbuf[t], acc_sp.at[col_v.at[j]], ssem[t]).wait()

            for k in range(EPR // LANES):
                w16 = ew_v[j, pl.ds(k * LANES, LANES)]
                for i in range(LANES):
                    e = k * LANES + i
                    sbuf[t][e, :] = gbuf[t][e, :] * _splat(w16, i)
            pltpu.async_copy(sbuf[t], acc_sp.at[col_v.at[j]], ssem[t], add=True)

            @pl.when(m < MAIN - 1)
            def _next_gather():
                pltpu.async_copy(
                    y_hbm.at[row_v.at[j + NBUF]], gbuf[t], gsem[t])
        return 0

    lax.fori_loop(0, MAIN, mbody, 0)
    for t in range(NBUF):
        jl = (MAIN - 1) * NBUF + t
        pltpu.make_async_copy(sbuf[t], acc_sp.at[col_v.at[jl]], ssem[t]).wait()
    plsc.subcore_barrier()
    pltpu.sync_copy(acc_sp.at[pl.ds(sid * NPT, NPT)], out_hbm.at[cid, sid])


def _tc_prep_body(dpt_ref, x_ref, w1_ref, y1_ref, dinv_ref):
    deg = jnp.sum(dpt_ref[...], axis=1, keepdims=True) + 1.0
    dinv = lax.rsqrt(deg)
    dinv_ref[...] = dinv
    y1_ref[...] = jnp.dot(x_ref[...], w1_ref[...],
                          preferred_element_type=jnp.float32) * dinv


def _tc_mid_body(a0_ref, a1_ref, y1_ref, dinv_ref, b1_ref, w2_ref, y2_ref):
    dinv = dinv_ref[...]
    h = (a0_ref[...] + a1_ref[...] + y1_ref[...]) * dinv + b1_ref[...]
    h = jnp.maximum(h, 0.0)
    y2_ref[...] = jnp.dot(h, w2_ref[...],
                          preferred_element_type=jnp.float32) * dinv


def _tc_final_body(a0_ref, a1_ref, y2_ref, dinv_ref, b2_ref, out_ref):
    z = (a0_ref[...] + a1_ref[...] + y2_ref[...]) * dinv_ref[...] + b2_ref[...]
    m = jnp.max(z, axis=1, keepdims=True)
    lse = jnp.log(jnp.sum(jnp.exp(z - m), axis=1, keepdims=True)) + m
    out_ref[...] = z - lse


def kernel(x, edge_index, edges_weight, W1, b1, W2, b2):
    row2d = edge_index[0].reshape(NW, RPT, EPR)
    col2d = edge_index[1].reshape(NW, RPT, EPR)
    ew2d = edges_weight.reshape(NW, RPT, EPR)

    deg_part = _sc_deg(col2d, ew2d)
    dpt = deg_part.T

    y1, dinv = pl.pallas_call(
        _tc_prep_body,
        out_shape=(jax.ShapeDtypeStruct((N, DH), jnp.float32),
                   jax.ShapeDtypeStruct((N, 1), jnp.float32)),
    )(dpt, x, W1)

    acc = _sc_prop(y1, row2d, col2d, ew2d).reshape(2, N, DH)

    w2pad = jnp.zeros((DH, DH), jnp.float32).at[:, :NC].set(W2)
    b1r = b1.reshape(1, DH)
    y2 = pl.pallas_call(
        _tc_mid_body,
        out_shape=jax.ShapeDtypeStruct((N, DH), jnp.float32),
    )(acc[0], acc[1], y1, dinv, b1r, w2pad)

    acc2 = _sc_prop(y2, row2d, col2d, ew2d).reshape(2, N, DH)

    b2pad = jnp.full((1, DH), -1e30, jnp.float32).at[0, :NC].set(b2)
    outp = pl.pallas_call(
        _tc_final_body,
        out_shape=jax.ShapeDtypeStruct((N, DH), jnp.float32),
    )(acc2[0], acc2[1], y2, dinv, b2pad)

    return outp[:, :NC]
